# Initial kernel scaffold; baseline (speedup 1.0000x reference)
import jax
import jax.numpy as jnp
from jax.experimental import pallas as pl


def kernel(x, emb_table, pos_table):
    raise NotImplementedError("write your pallas kernel here")


# speedup vs baseline: 3.4871x; 3.4871x over previous
import functools

import jax
import jax.numpy as jnp
from jax import lax
from jax.experimental import pallas as pl
from jax.experimental.pallas import tpu as pltpu
from jax.experimental.pallas import tpu_sc as plsc

EMB = 32
SEQ = 200
NW = 32
CW = 800
LANES = 16


def _sc_embed(x_flat, emb_table, pos_table):
  total = x_flat.shape[0]
  per_w = total // NW
  nchunk = per_w // CW
  mesh = plsc.VectorSubcoreMesh(core_axis_name="c", subcore_axis_name="s")

  @functools.partial(
      pl.kernel,
      out_type=jax.ShapeDtypeStruct((total, EMB), jnp.float32),
      mesh=mesh,
      scratch_types=[
          pltpu.VMEM((SEQ, EMB), jnp.float32),
          pltpu.VMEM((CW,), jnp.int32),
          pltpu.VMEM((CW, EMB), jnp.float32),
          pltpu.SemaphoreType.DMA,
      ],
      compiler_params=pltpu.CompilerParams(use_tc_tiling_on_sc=False),
  )
  def k(table_hbm, idx_hbm, pos_hbm, out_hbm, pos_v, idx_v, rows_v, sem):
    wid = lax.axis_index("s") * 2 + lax.axis_index("c")
    pltpu.sync_copy(pos_hbm, pos_v)
    base0 = wid * per_w

    @pl.loop(0, nchunk)
    def _chunk(c):
      base = base0 + c * CW
      pltpu.sync_copy(idx_hbm.at[pl.ds(base, CW)], idx_v)
      pltpu.async_copy(table_hbm.at[idx_v], rows_v, sem).wait()

      @pl.loop(0, CW)
      def _row(r):
        p = lax.rem(r, SEQ)
        rows_v[r, pl.ds(0, LANES)] = (
            rows_v[r, pl.ds(0, LANES)] + pos_v[p, pl.ds(0, LANES)])
        rows_v[r, pl.ds(LANES, LANES)] = (
            rows_v[r, pl.ds(LANES, LANES)] + pos_v[p, pl.ds(LANES, LANES)])

      pltpu.sync_copy(rows_v, out_hbm.at[pl.ds(base, CW)])

  return k(emb_table, x_flat, pos_table)


@jax.jit
def kernel(x, emb_table, pos_table):
  b, s = x.shape
  out = _sc_embed(x.reshape(-1), emb_table, pos_table)
  return out.reshape(b, s, EMB)


# speedup vs baseline: 5.2120x; 1.4946x over previous
import functools

import jax
import jax.numpy as jnp
from jax import lax
from jax.experimental import pallas as pl
from jax.experimental.pallas import tpu as pltpu
from jax.experimental.pallas import tpu_sc as plsc

EMB = 32
SEQ = 200
NW = 32
CW = 1600
REP = CW // SEQ
LANES = 16


def _sc_embed(x_flat, emb_table, pos_table):
  total = x_flat.shape[0]
  per_w = total // NW
  nchunk = per_w // CW
  mesh = plsc.VectorSubcoreMesh(core_axis_name="c", subcore_axis_name="s")

  @functools.partial(
      pl.kernel,
      out_type=jax.ShapeDtypeStruct((total, EMB), jnp.float32),
      mesh=mesh,
      scratch_types=[
          pltpu.VMEM((SEQ, EMB), jnp.float32),
          pltpu.VMEM((CW,), jnp.int32),
          pltpu.VMEM((CW,), jnp.int32),
          pltpu.VMEM((2, CW, EMB), jnp.float32),
          pltpu.SemaphoreType.DMA,
          pltpu.SemaphoreType.DMA,
          pltpu.SemaphoreType.DMA,
          pltpu.SemaphoreType.DMA,
      ],
      compiler_params=pltpu.CompilerParams(use_tc_tiling_on_sc=False),
  )
  def k(table_hbm, idx_hbm, pos_hbm, out_hbm, pos_v, idx_v0, idx_v1, rows_v,
        gsem0, gsem1, osem0, osem1):
    gsems = (gsem0, gsem1)
    osems = (osem0, osem1)
    idxs = (idx_v0, idx_v1)
    wid = lax.axis_index("s") * 2 + lax.axis_index("c")
    pltpu.sync_copy(pos_hbm, pos_v)
    base0 = wid * per_w

    pltpu.sync_copy(idx_hbm.at[pl.ds(base0, CW)], idx_v0)
    pltpu.async_copy(table_hbm.at[idx_v0], rows_v.at[0], gsems[0])

    @pl.loop(0, nchunk, step=2)
    def _pair(g):
      for off in range(2):
        b = off
        nb = 1 - b
        c = g + off
        base = base0 + c * CW

        @pl.when(c + 1 < nchunk)
        def _start_next():
          @pl.when(c >= 1)
          def _drain_out():
            pltpu.make_async_copy(
                rows_v.at[nb], out_hbm.at[pl.ds(base0, CW)], osems[nb]).wait()
          pltpu.sync_copy(idx_hbm.at[pl.ds(base + CW, CW)], idxs[nb])
          pltpu.async_copy(table_hbm.at[idxs[nb]], rows_v.at[nb], gsems[nb])

        pltpu.make_async_copy(
            table_hbm.at[idxs[b]], rows_v.at[b], gsems[b]).wait()

        @pl.loop(0, SEQ)
        def _pos(p):
          pa = pos_v[p, pl.ds(0, LANES)]
          pb = pos_v[p, pl.ds(LANES, LANES)]
          for rep in range(REP):
            r = rep * SEQ + p
            rows_v[b, r, pl.ds(0, LANES)] = rows_v[b, r, pl.ds(0, LANES)] + pa
            rows_v[b, r, pl.ds(LANES, LANES)] = (
                rows_v[b, r, pl.ds(LANES, LANES)] + pb)

        pltpu.async_copy(rows_v.at[b], out_hbm.at[pl.ds(base, CW)], osems[b])

    pltpu.make_async_copy(
        rows_v.at[0], out_hbm.at[pl.ds(base0, CW)], osems[0]).wait()
    pltpu.make_async_copy(
        rows_v.at[1], out_hbm.at[pl.ds(base0, CW)], osems[1]).wait()

  return k(emb_table, x_flat, pos_table)


@jax.jit
def kernel(x, emb_table, pos_table):
  b, s = x.shape
  out = _sc_embed(x.reshape(-1), emb_table, pos_table)
  return out.reshape(b, s, EMB)
